# baseline (device time: 188086 ns/iter reference)
import jax
import jax.numpy as jnp
from jax import lax
from jax.experimental import pallas as pl
from jax.experimental.pallas import tpu as pltpu

M, D = 8192, 2048
NQ = 4
Q = M // NQ
NC = 16
CH = Q // NC


def kernel(partial, resid, gamma):
    p = partial[0]
    g = gamma.reshape(1, D)

    def body(p_ref, resid_ref, g_ref, out_ref,
             pf_vm, pb_vm, res_vm, zrecv_vm, ob_vm,
             z_send, z_recv, x_send, x_recv, y_send, y_recv, loc):
        my_x = lax.axis_index("x")
        my_y = lax.axis_index("y")
        my_z = lax.axis_index("z")
        zp = (my_x, my_y, 1 - my_z)
        xp = (1 - my_x, my_y, my_z)
        yp = (my_x, 1 - my_y, my_z)

        q0 = 2 * my_x + my_y
        qx = 2 * (1 - my_x) + my_y
        qy = 2 * my_x + (1 - my_y)
        row0 = q0 * Q

        cp_res = pltpu.make_async_copy(
            resid_ref.at[pl.ds(row0, Q), :], res_vm, loc.at[1])
        cp_res.start()
        pf_loads = [pltpu.make_async_copy(
            p_ref.at[pl.ds(row0, CH), :], pf_vm.at[0], loc.at[2])]
        pf_loads[0].start()

        barrier = pltpu.get_barrier_semaphore()
        for nbr in (zp, xp, yp):
            pl.semaphore_signal(barrier, inc=1, device_id=nbr,
                                device_id_type=pl.DeviceIdType.MESH)
        pl.semaphore_wait(barrier, 3)

        z_rdmas = []
        for j in range(NC):
            pf_loads[j].wait()
            if j + 1 < NC:
                nxt = pltpu.make_async_copy(
                    p_ref.at[pl.ds(row0 + (j + 1) * CH, CH), :],
                    pf_vm.at[(j + 1) % 2], loc.at[2 + (j + 1) % 2])
                nxt.start()
                pf_loads.append(nxt)
            csl = pl.ds(j * CH, CH)
            pb_vm[csl, :] = pf_vm[j % 2].astype(jnp.bfloat16)
            rdma = pltpu.make_async_remote_copy(
                src_ref=pb_vm.at[csl, :],
                dst_ref=zrecv_vm.at[csl, :],
                send_sem=z_send.at[j],
                recv_sem=z_recv.at[j],
                device_id=zp,
                device_id_type=pl.DeviceIdType.MESH,
            )
            rdma.start()
            z_rdmas.append(rdma)

        cp_res.wait()

        x_rdmas = []
        y_rdmas = []
        for j in range(NC):
            csl = pl.ds(j * CH, CH)
            gsl = pl.ds(row0 + j * CH, CH)
            z_rdmas[j].wait_recv()
            y = (pb_vm[csl, :].astype(jnp.float32)
                 + zrecv_vm[csl, :].astype(jnp.float32)
                 + res_vm[csl, :])
            ms = jnp.mean(y * y, axis=-1, keepdims=True)
            ob_vm[csl, :] = (y * lax.rsqrt(ms + 1e-6)
                             * g_ref[...]).astype(jnp.bfloat16)
            cp_o = pltpu.make_async_copy(
                ob_vm.at[csl, :], out_ref.at[gsl, :], loc.at[0])
            cp_o.start()
            for partner, sems_s, sems_r, lst in (
                    (xp, x_send, x_recv, x_rdmas),
                    (yp, y_send, y_recv, y_rdmas)):
                rdma = pltpu.make_async_remote_copy(
                    src_ref=ob_vm.at[csl, :],
                    dst_ref=out_ref.at[gsl, :],
                    send_sem=sems_s.at[j],
                    recv_sem=sems_r.at[j],
                    device_id=partner,
                    device_id_type=pl.DeviceIdType.MESH,
                )
                rdma.start()
                lst.append(rdma)
            cp_o.wait()

        H = NC // 2
        for j in range(H):
            gsl = pl.ds(qx * Q + j * CH, CH)
            x_rdmas[j].wait_recv()
            fwd = pltpu.make_async_remote_copy(
                src_ref=out_ref.at[gsl, :],
                dst_ref=out_ref.at[gsl, :],
                send_sem=y_send.at[NC + j],
                recv_sem=y_recv.at[NC + j],
                device_id=yp,
                device_id_type=pl.DeviceIdType.MESH,
            )
            fwd.start()
            y_rdmas.append(fwd)
        for j in range(H, NC):
            gsl = pl.ds(qy * Q + j * CH, CH)
            y_rdmas[j].wait_recv()
            fwd = pltpu.make_async_remote_copy(
                src_ref=out_ref.at[gsl, :],
                dst_ref=out_ref.at[gsl, :],
                send_sem=x_send.at[NC + j - H],
                recv_sem=x_recv.at[NC + j - H],
                device_id=xp,
                device_id_type=pl.DeviceIdType.MESH,
            )
            fwd.start()
            x_rdmas.append(fwd)

        for k in list(range(H, NC)) + list(range(NC, NC + H)):
            x_rdmas[k].wait_recv()
        for k in list(range(H)) + list(range(NC, NC + H)):
            y_rdmas[k].wait_recv()

        for rdma in z_rdmas + x_rdmas + y_rdmas:
            rdma.wait_send()

    return pl.pallas_call(
        body,
        out_shape=jax.ShapeDtypeStruct((M, D), jnp.bfloat16),
        in_specs=[
            pl.BlockSpec(memory_space=pl.ANY),
            pl.BlockSpec(memory_space=pl.ANY),
            pl.BlockSpec(memory_space=pltpu.VMEM),
        ],
        out_specs=pl.BlockSpec(memory_space=pl.ANY),
        scratch_shapes=[
            pltpu.VMEM((2, CH, D), jnp.float32),
            pltpu.VMEM((Q, D), jnp.bfloat16),
            pltpu.VMEM((Q, D), jnp.float32),
            pltpu.VMEM((Q, D), jnp.bfloat16),
            pltpu.VMEM((Q, D), jnp.bfloat16),
            pltpu.SemaphoreType.DMA((NC,)),
            pltpu.SemaphoreType.DMA((NC,)),
            pltpu.SemaphoreType.DMA((NC + NC // 2,)),
            pltpu.SemaphoreType.DMA((NC + NC // 2,)),
            pltpu.SemaphoreType.DMA((NC + NC // 2,)),
            pltpu.SemaphoreType.DMA((NC + NC // 2,)),
            pltpu.SemaphoreType.DMA((4,)),
        ],
        compiler_params=pltpu.CompilerParams(
            collective_id=0, vmem_limit_bytes=60 * 1024 * 1024),
    )(p, resid, g)


# device time: 183006 ns/iter; 1.0278x vs baseline; 1.0278x over previous
import jax
import jax.numpy as jnp
from jax import lax
from jax.experimental import pallas as pl
from jax.experimental.pallas import tpu as pltpu

M, D = 8192, 2048
NQ = 4
Q = M // NQ
NC = 8
CH = Q // NC


def kernel(partial, resid, gamma):
    p = partial[0]
    g = gamma.reshape(1, D)

    def body(p_ref, resid_ref, g_ref, out_ref,
             pf_vm, pb_vm, res_vm, zrecv_vm, ob_vm,
             z_send, z_recv, x_send, x_recv, y_send, y_recv, loc):
        my_x = lax.axis_index("x")
        my_y = lax.axis_index("y")
        my_z = lax.axis_index("z")
        zp = (my_x, my_y, 1 - my_z)
        xp = (1 - my_x, my_y, my_z)
        yp = (my_x, 1 - my_y, my_z)

        q0 = 2 * my_x + my_y
        qx = 2 * (1 - my_x) + my_y
        qy = 2 * my_x + (1 - my_y)
        row0 = q0 * Q

        cp_res = pltpu.make_async_copy(
            resid_ref.at[pl.ds(row0, Q), :], res_vm, loc.at[1])
        cp_res.start()
        pf_loads = [pltpu.make_async_copy(
            p_ref.at[pl.ds(row0, CH), :], pf_vm.at[0], loc.at[2])]
        pf_loads[0].start()

        barrier = pltpu.get_barrier_semaphore()
        for nbr in (zp, xp, yp):
            pl.semaphore_signal(barrier, inc=1, device_id=nbr,
                                device_id_type=pl.DeviceIdType.MESH)
        pl.semaphore_wait(barrier, 3)

        z_rdmas = []
        for j in range(NC):
            pf_loads[j].wait()
            if j + 1 < NC:
                nxt = pltpu.make_async_copy(
                    p_ref.at[pl.ds(row0 + (j + 1) * CH, CH), :],
                    pf_vm.at[(j + 1) % 2], loc.at[2 + (j + 1) % 2])
                nxt.start()
                pf_loads.append(nxt)
            csl = pl.ds(j * CH, CH)
            pb_vm[csl, :] = pf_vm[j % 2].astype(jnp.bfloat16)
            rdma = pltpu.make_async_remote_copy(
                src_ref=pb_vm.at[csl, :],
                dst_ref=zrecv_vm.at[csl, :],
                send_sem=z_send.at[j],
                recv_sem=z_recv.at[j],
                device_id=zp,
                device_id_type=pl.DeviceIdType.MESH,
            )
            rdma.start()
            z_rdmas.append(rdma)

        cp_res.wait()

        x_rdmas = []
        y_rdmas = []
        for j in range(NC):
            csl = pl.ds(j * CH, CH)
            gsl = pl.ds(row0 + j * CH, CH)
            z_rdmas[j].wait_recv()
            y = (pb_vm[csl, :].astype(jnp.float32)
                 + zrecv_vm[csl, :].astype(jnp.float32)
                 + res_vm[csl, :])
            ms = jnp.mean(y * y, axis=-1, keepdims=True)
            ob_vm[csl, :] = (y * lax.rsqrt(ms + 1e-6)
                             * g_ref[...]).astype(jnp.bfloat16)
            for partner, sems_s, sems_r, lst in (
                    (xp, x_send, x_recv, x_rdmas),
                    (yp, y_send, y_recv, y_rdmas)):
                rdma = pltpu.make_async_remote_copy(
                    src_ref=ob_vm.at[csl, :],
                    dst_ref=out_ref.at[gsl, :],
                    send_sem=sems_s.at[j],
                    recv_sem=sems_r.at[j],
                    device_id=partner,
                    device_id_type=pl.DeviceIdType.MESH,
                )
                rdma.start()
                lst.append(rdma)

        cp_o = pltpu.make_async_copy(
            ob_vm, out_ref.at[pl.ds(row0, Q), :], loc.at[0])
        cp_o.start()

        H = NC // 2
        for j in range(H):
            gsl = pl.ds(qx * Q + j * CH, CH)
            x_rdmas[j].wait_recv()
            fwd = pltpu.make_async_remote_copy(
                src_ref=out_ref.at[gsl, :],
                dst_ref=out_ref.at[gsl, :],
                send_sem=y_send.at[NC + j],
                recv_sem=y_recv.at[NC + j],
                device_id=yp,
                device_id_type=pl.DeviceIdType.MESH,
            )
            fwd.start()
            y_rdmas.append(fwd)
        for j in range(H, NC):
            gsl = pl.ds(qy * Q + j * CH, CH)
            y_rdmas[j].wait_recv()
            fwd = pltpu.make_async_remote_copy(
                src_ref=out_ref.at[gsl, :],
                dst_ref=out_ref.at[gsl, :],
                send_sem=x_send.at[NC + j - H],
                recv_sem=x_recv.at[NC + j - H],
                device_id=xp,
                device_id_type=pl.DeviceIdType.MESH,
            )
            fwd.start()
            x_rdmas.append(fwd)

        for k in list(range(H, NC)) + list(range(NC, NC + H)):
            x_rdmas[k].wait_recv()
        for k in list(range(H)) + list(range(NC, NC + H)):
            y_rdmas[k].wait_recv()

        cp_o.wait()

        for rdma in z_rdmas + x_rdmas + y_rdmas:
            rdma.wait_send()

    return pl.pallas_call(
        body,
        out_shape=jax.ShapeDtypeStruct((M, D), jnp.bfloat16),
        in_specs=[
            pl.BlockSpec(memory_space=pl.ANY),
            pl.BlockSpec(memory_space=pl.ANY),
            pl.BlockSpec(memory_space=pltpu.VMEM),
        ],
        out_specs=pl.BlockSpec(memory_space=pl.ANY),
        scratch_shapes=[
            pltpu.VMEM((2, CH, D), jnp.float32),
            pltpu.VMEM((Q, D), jnp.bfloat16),
            pltpu.VMEM((Q, D), jnp.float32),
            pltpu.VMEM((Q, D), jnp.bfloat16),
            pltpu.VMEM((Q, D), jnp.bfloat16),
            pltpu.SemaphoreType.DMA((NC,)),
            pltpu.SemaphoreType.DMA((NC,)),
            pltpu.SemaphoreType.DMA((NC + NC // 2,)),
            pltpu.SemaphoreType.DMA((NC + NC // 2,)),
            pltpu.SemaphoreType.DMA((NC + NC // 2,)),
            pltpu.SemaphoreType.DMA((NC + NC // 2,)),
            pltpu.SemaphoreType.DMA((4,)),
        ],
        compiler_params=pltpu.CompilerParams(
            collective_id=0, vmem_limit_bytes=60 * 1024 * 1024),
    )(p, resid, g)
